# baseline (speedup 1.0000x reference)
import functools

import jax
import jax.numpy as jnp
from jax import lax
from jax.experimental import pallas as pl
from jax.experimental.pallas import tpu as pltpu

_R = 16
_K = 4


def _tc_add(x2, pos2):
    batch, slab = x2.shape
    nchunk = batch // _R
    rounds = nchunk // _K

    def body(pos_ref, x_hbm, o_hbm, ibuf, obuf, insem, outsem):
        def start_in(c_idx, k):
            pltpu.make_async_copy(
                x_hbm.at[pl.ds(c_idx * _R, _R)], ibuf.at[k], insem.at[k]
            ).start(priority=k % 2)

        def wait_in(c_idx, k):
            pltpu.make_async_copy(
                x_hbm.at[pl.ds(c_idx * _R, _R)], ibuf.at[k], insem.at[k]
            ).wait()

        def start_out(c_idx, k):
            pltpu.make_async_copy(
                obuf.at[k], o_hbm.at[pl.ds(c_idx * _R, _R)], outsem.at[k]
            ).start(priority=k % 2)

        def wait_out(c_idx, k):
            pltpu.make_async_copy(
                obuf.at[k], o_hbm.at[pl.ds(c_idx * _R, _R)], outsem.at[k]
            ).wait()

        def compute(k):
            obuf[k] = ibuf[k] + pos_ref[...]

        for k in range(_K):
            start_in(k, k)

        for k in range(_K):
            wait_in(k, k)
            compute(k)
            start_out(k, k)
            start_in(_K + k, k)

        def round_body(t, carry):
            for k in range(_K):
                c = t * _K + k
                wait_in(c, k)
                wait_out(c - _K, k)
                compute(k)
                start_out(c, k)
                start_in(c + _K, k)
            return carry

        lax.fori_loop(1, rounds - 1, round_body, 0)

        t_last = rounds - 1
        for k in range(_K):
            c = t_last * _K + k
            wait_in(c, k)
            wait_out(c - _K, k)
            compute(k)
            start_out(c, k)
        for k in range(_K):
            wait_out(t_last * _K + k, k)

    return pl.pallas_call(
        body,
        in_specs=[
            pl.BlockSpec(memory_space=pltpu.MemorySpace.VMEM),
            pl.BlockSpec(memory_space=pl.ANY),
        ],
        out_specs=pl.BlockSpec(memory_space=pl.ANY),
        out_shape=jax.ShapeDtypeStruct((batch, slab), jnp.float32),
        scratch_shapes=[
            pltpu.VMEM((_K, _R, slab), jnp.float32),
            pltpu.VMEM((_K, _R, slab), jnp.float32),
            pltpu.SemaphoreType.DMA((_K,)),
            pltpu.SemaphoreType.DMA((_K,)),
        ],
    )(pos2, x2)


def kernel(x, pos_emb):
    batch, maxlen, dim = x.shape
    slab = maxlen * dim
    x2 = x.reshape(batch, slab)
    pos2 = pos_emb.reshape(1, slab)
    out2 = _tc_add(x2, pos2)
    return out2.reshape(batch, maxlen, dim)


# speedup vs baseline: 3.5003x; 3.5003x over previous
import jax
import jax.numpy as jnp
from jax.experimental import pallas as pl
from jax.experimental.pallas import tpu as pltpu


def _tc_add(x, pos, b_blk=32):
    batch, maxlen, dim = x.shape

    def body(x_ref, pos_ref, o_ref):
        o_ref[...] = x_ref[...] + pos_ref[...][None]

    return pl.pallas_call(
        body,
        grid=(batch // b_blk,),
        in_specs=[
            pl.BlockSpec((b_blk, maxlen, dim), lambda i: (i, 0, 0)),
            pl.BlockSpec((maxlen, dim), lambda i: (0, 0)),
        ],
        out_specs=pl.BlockSpec((b_blk, maxlen, dim), lambda i: (i, 0, 0)),
        out_shape=jax.ShapeDtypeStruct((batch, maxlen, dim), jnp.float32),
    )(x, pos)


def kernel(x, pos_emb):
    return _tc_add(x, pos_emb)
